# Initial kernel scaffold; baseline (speedup 1.0000x reference)
import jax
import jax.numpy as jnp
from jax.experimental import pallas as pl


def kernel(x, edge_index, edge_attr, batch, node_embeddings, W1, b1, att_src1, att_dst1, We1, att_e1, W2, b2, att_src2, att_dst2, We2, att_e2, Wfc, bfc):
    raise NotImplementedError("write your pallas kernel here")


# speedup vs baseline: 16.0751x; 16.0751x over previous
import functools

import jax
import jax.numpy as jnp
from jax import lax
from jax.experimental import pallas as pl
from jax.experimental.pallas import tpu as pltpu
from jax.experimental.pallas import tpu_sc as plsc

N = 10000
E = 320000
D = 128
DE = 4
VOCAB = 1000

NC = 2
NS = 16
LANES = 16
NW = NC * NS
EPT = E // NW
SUB = 80
ROWS_PER_CHUNK = 5
CK = SUB * ROWS_PER_CHUNK
NCHUNK = EPT // CK
ROWS_TOTAL = E // SUB
ROWS_PT = ROWS_TOTAL // NW
ZROWS = 125
NPT = N // NS

_f32 = jnp.float32
_i32 = jnp.int32


def _sc_mesh():
    return plsc.VectorSubcoreMesh(core_axis_name="c", subcore_axis_name="s")


_SC_PARAMS = pltpu.CompilerParams(use_tc_tiling_on_sc=False,
                                  needs_layout_passes=False)


def _erf(z):
    a1, a2, a3, a4, a5 = (0.254829592, -0.284496736, 1.421413741,
                          -1.453152027, 1.061405429)
    p = 0.3275911
    s = jnp.sign(z)
    x = jnp.abs(z)
    t = 1.0 / (1.0 + p * x)
    poly = t * (a1 + t * (a2 + t * (a3 + t * (a4 + t * a5))))
    return s * (1.0 - poly * jnp.exp(-x * x))


def _gelu(z):
    return 0.5 * z * (1.0 + lax.erf(z * (2.0 ** -0.5)))


def _edgeprep_body(ea_ref, we1_ref, ae1_ref, we2_ref, ae2_ref,
                   et1_ref, et2_ref, msum_ref):
    i = pl.program_id(0)
    ea = ea_ref[...]
    wv1 = jnp.sum(we1_ref[...] * ae1_ref[...], axis=1)
    wv2 = jnp.sum(we2_ref[...] * ae2_ref[...], axis=1)
    et1_ref[0, 0, :] = jnp.sum(ea * wv1[None, :], axis=1)
    et2_ref[0, 0, :] = jnp.sum(ea * wv2[None, :], axis=1)

    @pl.when(i == 0)
    def _():
        msum_ref[...] = jnp.zeros_like(msum_ref)
    msum_ref[0, :] += jnp.sum(ea, axis=0)


def _edgeprep(edge_attr, We1, att_e1, We2, att_e2):
    BE = 8000
    G = E // BE
    return pl.pallas_call(
        _edgeprep_body,
        grid=(G,),
        in_specs=[
            pl.BlockSpec((BE, DE), lambda i: (i, 0)),
            pl.BlockSpec((DE, D), lambda i: (0, 0)),
            pl.BlockSpec((1, D), lambda i: (0, 0)),
            pl.BlockSpec((DE, D), lambda i: (0, 0)),
            pl.BlockSpec((1, D), lambda i: (0, 0)),
        ],
        out_specs=[
            pl.BlockSpec((1, 1, BE), lambda i: (i, 0, 0)),
            pl.BlockSpec((1, 1, BE), lambda i: (i, 0, 0)),
            pl.BlockSpec((1, DE), lambda i: (0, 0)),
        ],
        out_shape=[
            jax.ShapeDtypeStruct((G, 1, BE), _f32),
            jax.ShapeDtypeStruct((G, 1, BE), _f32),
            jax.ShapeDtypeStruct((1, DE), _f32),
        ],
    )(edge_attr, We1, att_e1[None, :], We2, att_e2[None, :])


def _nodeprep_body(x_ref, tab_ref, w1_ref, as1_ref, ad1_ref,
                   h1_ref, asrc_ref, adst_ref):
    xb = x_ref[0, 0, :]
    onehot = (xb[:, None] == lax.broadcasted_iota(_i32, (1, VOCAB), 1)
              ).astype(_f32)
    h0 = jnp.dot(onehot, tab_ref[...], preferred_element_type=_f32,
                 precision=lax.Precision.HIGHEST)
    h1 = jnp.dot(h0, w1_ref[...], preferred_element_type=_f32,
                 precision=lax.Precision.HIGHEST)
    h1_ref[...] = h1
    asrc_ref[0, 0, :] = jnp.sum(h1 * as1_ref[...], axis=1)
    adst_ref[0, 0, :] = jnp.sum(h1 * ad1_ref[...], axis=1)


def _nodeprep(x, table, W1, att_src1, att_dst1):
    BN = 1000
    G = N // BN
    return pl.pallas_call(
        _nodeprep_body,
        grid=(G,),
        in_specs=[
            pl.BlockSpec((1, 1, BN), lambda i: (i, 0, 0)),
            pl.BlockSpec((VOCAB, D), lambda i: (0, 0)),
            pl.BlockSpec((D, D), lambda i: (0, 0)),
            pl.BlockSpec((1, D), lambda i: (0, 0)),
            pl.BlockSpec((1, D), lambda i: (0, 0)),
        ],
        out_specs=[
            pl.BlockSpec((BN, D), lambda i: (i, 0)),
            pl.BlockSpec((1, 1, BN), lambda i: (i, 0, 0)),
            pl.BlockSpec((1, 1, BN), lambda i: (i, 0, 0)),
        ],
        out_shape=[
            jax.ShapeDtypeStruct((N, D), _f32),
            jax.ShapeDtypeStruct((G, 1, BN), _f32),
            jax.ShapeDtypeStruct((G, 1, BN), _f32),
        ],
    )(x.reshape(G, 1, BN), table, W1, att_src1[None, :], att_dst1[None, :])


def _gat_edge_sc(src_hbm, dst_hbm, et_hbm, h_hbm, asrc_hbm, adst_hbm,
                 numer_out,
                 asrc_v, adst_v, src_b, dst_b, et_b, rows, ex_b,
                 zbuf, numer_s, gsem):
    c = lax.axis_index("c")
    s = lax.axis_index("s")
    wid = s * NC + c

    pltpu.sync_copy(asrc_hbm, asrc_v)
    pltpu.sync_copy(adst_hbm, adst_v)

    zero16 = jnp.zeros((LANES,), _f32)

    @pl.loop(0, ZROWS)
    def _(r):
        for v in range(D // LANES):
            zbuf[r, pl.ds(v * LANES, LANES)] = zero16

    @pl.loop(0, SUB)
    def _(r):
        ex_b[r, pl.ds(0, LANES)] = zero16

    n0 = s * NPT
    for z in range(NPT // ZROWS):
        pltpu.sync_copy(zbuf, numer_s.at[pl.ds(n0 + z * ZROWS, ZROWS)])
    plsc.subcore_barrier()

    row0 = wid * ROWS_PT

    @pl.loop(0, NCHUNK)
    def _(ch):
        r0 = row0 + ch * ROWS_PER_CHUNK
        pltpu.sync_copy(src_hbm.at[pl.ds(r0, ROWS_PER_CHUNK)], src_b)
        pltpu.sync_copy(dst_hbm.at[pl.ds(r0, ROWS_PER_CHUNK)], dst_b)
        pltpu.sync_copy(et_hbm.at[pl.ds(r0, ROWS_PER_CHUNK)], et_b)

        for j in range(ROWS_PER_CHUNK):
            gcopy = pltpu.async_copy(h_hbm.at[src_b.at[j]], rows, gsem)

            for g in range(SUB // LANES):
                sl = pl.ds(g * LANES, LANES)
                s16 = src_b[j, sl]
                d16 = dst_b[j, sl]
                asv = plsc.load_gather(asrc_v, [s16])
                adv = plsc.load_gather(adst_v, [d16])
                al = asv + adv + et_b[j, sl]
                al = jnp.maximum(al, 0.2 * al)
                ex = jnp.exp(al)
                ridx = lax.iota(_i32, LANES) + (g * LANES)
                plsc.store_scatter(ex_b, [ridx, jnp.zeros((LANES,), _i32)], ex)

            gcopy.wait()

            @pl.loop(0, SUB)
            def _(e):
                w = ex_b[e, pl.ds(0, LANES)][0]
                for v in range(D // LANES):
                    sl = pl.ds(v * LANES, LANES)
                    rows[e, sl] = rows[e, sl] * w

            pltpu.sync_copy(rows, numer_s.at[dst_b.at[j]], add=True)

    plsc.subcore_barrier()

    for z in range(NPT // ZROWS):
        base = n0 + z * ZROWS
        pltpu.sync_copy(numer_s.at[pl.ds(base, ZROWS)],
                        numer_out.at[c, pl.ds(base, ZROWS)])


def _gat_edge(src2d, dst2d, et2d, h, asrc, adst):
    kern = pl.kernel(
        _gat_edge_sc,
        out_type=jax.ShapeDtypeStruct((NC, N, D), _f32),
        mesh=_sc_mesh(),
        scratch_types=[
            pltpu.VMEM((N,), _f32),
            pltpu.VMEM((N,), _f32),
            pltpu.VMEM((ROWS_PER_CHUNK, SUB), _i32),
            pltpu.VMEM((ROWS_PER_CHUNK, SUB), _i32),
            pltpu.VMEM((ROWS_PER_CHUNK, SUB), _f32),
            pltpu.VMEM((SUB, D), _f32),
            pltpu.VMEM((SUB, LANES), _f32),
            pltpu.VMEM((ZROWS, D), _f32),
            pltpu.VMEM_SHARED((N, D), _f32),
            pltpu.SemaphoreType.DMA,
        ],
        compiler_params=_SC_PARAMS,
    )
    return kern(src2d, dst2d, et2d, h, asrc, adst)


def _gat_den_sc(src_hbm, dst_hbm, et_hbm, asrc_hbm, adst_hbm,
                den_out,
                asrc_v, adst_v, src_b, dst_b, et_b, ex_b, dzbuf, den_s):
    c = lax.axis_index("c")
    s = lax.axis_index("s")
    wid = s * NC + c

    pltpu.sync_copy(asrc_hbm, asrc_v)
    pltpu.sync_copy(adst_hbm, adst_v)

    zero16 = jnp.zeros((LANES,), _f32)

    @pl.loop(0, ZROWS)
    def _(r):
        dzbuf[r, pl.ds(0, LANES)] = zero16

    @pl.loop(0, SUB)
    def _(r):
        ex_b[r, pl.ds(0, LANES)] = zero16

    n0 = s * NPT
    for z in range(NPT // ZROWS):
        pltpu.sync_copy(dzbuf, den_s.at[pl.ds(n0 + z * ZROWS, ZROWS)])
    plsc.subcore_barrier()

    row0 = wid * ROWS_PT

    @pl.loop(0, NCHUNK)
    def _(ch):
        r0 = row0 + ch * ROWS_PER_CHUNK
        pltpu.sync_copy(src_hbm.at[pl.ds(r0, ROWS_PER_CHUNK)], src_b)
        pltpu.sync_copy(dst_hbm.at[pl.ds(r0, ROWS_PER_CHUNK)], dst_b)
        pltpu.sync_copy(et_hbm.at[pl.ds(r0, ROWS_PER_CHUNK)], et_b)

        for j in range(ROWS_PER_CHUNK):
            for g in range(SUB // LANES):
                sl = pl.ds(g * LANES, LANES)
                asv = plsc.load_gather(asrc_v, [src_b[j, sl]])
                adv = plsc.load_gather(adst_v, [dst_b[j, sl]])
                al = asv + adv + et_b[j, sl]
                al = jnp.maximum(al, 0.2 * al)
                ex = jnp.exp(al)
                ridx = lax.iota(_i32, LANES) + (g * LANES)
                plsc.store_scatter(ex_b, [ridx, jnp.zeros((LANES,), _i32)], ex)
            pltpu.sync_copy(ex_b, den_s.at[dst_b.at[j]], add=True)

    plsc.subcore_barrier()

    for z in range(NPT // ZROWS):
        base = n0 + z * ZROWS
        pltpu.sync_copy(den_s.at[pl.ds(base, ZROWS)],
                        den_out.at[c, pl.ds(base, ZROWS)])


def _gat_den(src2d, dst2d, et2d, asrc, adst):
    kern = pl.kernel(
        _gat_den_sc,
        out_type=jax.ShapeDtypeStruct((NC, N, LANES), _f32),
        mesh=_sc_mesh(),
        scratch_types=[
            pltpu.VMEM((N,), _f32),
            pltpu.VMEM((N,), _f32),
            pltpu.VMEM((ROWS_PER_CHUNK, SUB), _i32),
            pltpu.VMEM((ROWS_PER_CHUNK, SUB), _i32),
            pltpu.VMEM((ROWS_PER_CHUNK, SUB), _f32),
            pltpu.VMEM((SUB, LANES), _f32),
            pltpu.VMEM((ZROWS, LANES), _f32),
            pltpu.VMEM_SHARED((N, LANES), _f32),
        ],
        compiler_params=_SC_PARAMS,
    )
    return kern(src2d, dst2d, et2d, asrc, adst)


def _finalize_body(np_ref, dp_ref, h_ref, asrc_ref, adst_ref, b_ref,
                   msum_ref, we_ref, ae_ref, wa_ref, wb_ref,
                   out_a_ref, out_b_ref, out_h_ref):
    wv = jnp.sum(we_ref[...] * ae_ref[...], axis=1)
    c_loop = jnp.sum(msum_ref[0, :] * wv) * (1.0 / E)
    sal = asrc_ref[0, 0, :] + adst_ref[0, 0, :] + c_loop
    sal = jnp.maximum(sal, 0.2 * sal)
    sex = jnp.exp(sal)
    numer = np_ref[0] + np_ref[1] + sex[:, None] * h_ref[...]
    denom = dp_ref[0, :, 0] + dp_ref[1, :, 0] + sex
    g = numer / (denom[:, None] + 1e-16) + b_ref[...]
    a = _gelu(g)
    ha = jnp.dot(a, wa_ref[...], preferred_element_type=_f32,
                 precision=lax.Precision.HIGHEST)
    hb = jnp.dot(a, wb_ref[...], preferred_element_type=_f32,
                 precision=lax.Precision.HIGHEST)
    out_h_ref[...] = ha
    out_a_ref[0, 0, :] = hb[:, 0]
    out_b_ref[0, 0, :] = hb[:, 1]


def _finalize(numerP, denP, h, asrc, adst, b, msum, We, att_e, Wa, Wb):
    BN = 1000
    G = N // BN
    return pl.pallas_call(
        _finalize_body,
        grid=(G,),
        in_specs=[
            pl.BlockSpec((NC, BN, D), lambda i: (0, i, 0)),
            pl.BlockSpec((NC, BN, LANES), lambda i: (0, i, 0)),
            pl.BlockSpec((BN, D), lambda i: (i, 0)),
            pl.BlockSpec((1, 1, BN), lambda i: (i, 0, 0)),
            pl.BlockSpec((1, 1, BN), lambda i: (i, 0, 0)),
            pl.BlockSpec((1, D), lambda i: (0, 0)),
            pl.BlockSpec((1, DE), lambda i: (0, 0)),
            pl.BlockSpec((DE, D), lambda i: (0, 0)),
            pl.BlockSpec((1, D), lambda i: (0, 0)),
            pl.BlockSpec((D, D), lambda i: (0, 0)),
            pl.BlockSpec((D, 2), lambda i: (0, 0)),
        ],
        out_specs=[
            pl.BlockSpec((1, 1, BN), lambda i: (i, 0, 0)),
            pl.BlockSpec((1, 1, BN), lambda i: (i, 0, 0)),
            pl.BlockSpec((BN, D), lambda i: (i, 0)),
        ],
        out_shape=[
            jax.ShapeDtypeStruct((G, 1, BN), _f32),
            jax.ShapeDtypeStruct((G, 1, BN), _f32),
            jax.ShapeDtypeStruct((N, D), _f32),
        ],
    )(numerP, denP, h, asrc, adst, b[None, :], msum, We, att_e[None, :],
      Wa, Wb)


def _edge_score_sc(src_hbm, dst_hbm, p_hbm, q_hbm, out_hbm,
                   p_v, q_v, src_b, dst_b, out_b):
    c = lax.axis_index("c")
    s = lax.axis_index("s")
    wid = s * NC + c
    pltpu.sync_copy(p_hbm, p_v)
    pltpu.sync_copy(q_hbm, q_v)
    row0 = wid * ROWS_PT

    @pl.loop(0, NCHUNK)
    def _(ch):
        r0 = row0 + ch * ROWS_PER_CHUNK
        pltpu.sync_copy(src_hbm.at[pl.ds(r0, ROWS_PER_CHUNK)], src_b)
        pltpu.sync_copy(dst_hbm.at[pl.ds(r0, ROWS_PER_CHUNK)], dst_b)
        for j in range(ROWS_PER_CHUNK):
            for g in range(SUB // LANES):
                sl = pl.ds(g * LANES, LANES)
                pv = plsc.load_gather(p_v, [src_b[j, sl]])
                qv = plsc.load_gather(q_v, [dst_b[j, sl]])
                out_b[j, sl] = pv + qv
        pltpu.sync_copy(out_b, out_hbm.at[pl.ds(r0, ROWS_PER_CHUNK)])


def _edge_score(src2d, dst2d, p, q):
    kern = pl.kernel(
        _edge_score_sc,
        out_type=jax.ShapeDtypeStruct((ROWS_TOTAL, SUB), _f32),
        mesh=_sc_mesh(),
        scratch_types=[
            pltpu.VMEM((N,), _f32),
            pltpu.VMEM((N,), _f32),
            pltpu.VMEM((ROWS_PER_CHUNK, SUB), _i32),
            pltpu.VMEM((ROWS_PER_CHUNK, SUB), _i32),
            pltpu.VMEM((ROWS_PER_CHUNK, SUB), _f32),
        ],
        compiler_params=_SC_PARAMS,
    )
    return kern(src2d, dst2d, p, q)


def _gelu_body(z_ref, o_ref):
    o_ref[...] = _gelu(z_ref[...])


def _gelu_edges(z2d):
    BE = 8000
    G = E // BE
    return pl.pallas_call(
        _gelu_body,
        grid=(G,),
        in_specs=[pl.BlockSpec((1, 1, BE), lambda i: (i, 0, 0))],
        out_specs=pl.BlockSpec((1, 1, BE), lambda i: (i, 0, 0)),
        out_shape=jax.ShapeDtypeStruct((G, 1, BE), _f32),
    )(z2d.reshape(G, 1, BE))


def kernel(x, edge_index, edge_attr, batch, node_embeddings,
           W1, b1, att_src1, att_dst1, We1, att_e1,
           W2, b2, att_src2, att_dst2, We2, att_e2, Wfc, bfc):
    del batch
    x = x.astype(_i32)
    edge_index = edge_index.astype(_i32)
    edge_attr = edge_attr.astype(_f32)

    src2d = edge_index[0].reshape(ROWS_TOTAL, SUB)
    dst2d = edge_index[1].reshape(ROWS_TOTAL, SUB)

    et1, et2, msum = _edgeprep(edge_attr, We1, att_e1, We2, att_e2)
    et1_2d = et1.reshape(ROWS_TOTAL, SUB)
    et2_2d = et2.reshape(ROWS_TOTAL, SUB)

    h1, asrc1, adst1 = _nodeprep(x, node_embeddings, W1, att_src1, att_dst1)

    numer1 = _gat_edge(src2d, dst2d, et1_2d, h1,
                       asrc1.reshape(N), adst1.reshape(N))
    den1 = _gat_den(src2d, dst2d, et1_2d,
                    asrc1.reshape(N), adst1.reshape(N))

    Wb1 = jnp.stack([att_src2, att_dst2], axis=1)
    asrc2, adst2, h2 = _finalize(numer1, den1, h1, asrc1, adst1, b1,
                                 msum, We1, att_e1, W2, Wb1)

    numer2 = _gat_edge(src2d, dst2d, et2_2d, h2,
                       asrc2.reshape(N), adst2.reshape(N))
    den2 = _gat_den(src2d, dst2d, et2_2d,
                    asrc2.reshape(N), adst2.reshape(N))

    Wfc_pair = Wfc.reshape(2, D, 1)
    Wb2 = jnp.concatenate([Wfc_pair[0], Wfc_pair[1]], axis=1)
    p, q, _unused = _finalize(numer2, den2, h2, asrc2, adst2, b2,
                              msum, We2, att_e2, jnp.zeros((D, D), _f32),
                              Wb2)
    p = (p + bfc[0]).reshape(N)
    q = q.reshape(N)

    scores = _edge_score(src2d, dst2d, p, q)
    out = _gelu_edges(scores)
    return out.reshape(E)
